# baseline (speedup 1.0000x reference)
import jax
import jax.numpy as jnp
from jax import lax
from jax.experimental import pallas as pl
from jax.experimental.pallas import tpu as pltpu

N = 320000
D = 128
NUM_SEG = 10000

R = 512
NBLK = N // R
NBLK_PAD = 640
S = 128
NW = (NUM_SEG + S - 1) // S
SEGP = NW * S
CHUNKS = NBLK + 2 * NW
CH_PAD = 896
BIG = 1 << 24


def _schedule_kernel(bf_ref, bl_ref, rb_ref, wc_ref, fl_ref):
    hi = lax.Precision.HIGHEST
    bf = bf_ref[...].astype(jnp.float32)
    bl = bl_ref[...].astype(jnp.float32)
    lane = lax.broadcasted_iota(jnp.int32, (1, S), 1)
    wvalid = lane < NW
    wvf = wvalid.astype(jnp.float32)
    bnds = (lane * S).astype(jnp.float32)

    ones_b = jnp.ones((1, NBLK_PAD), jnp.float32)
    b0 = jnp.dot(ones_b, (bl < bnds).astype(jnp.float32),
                 preferred_element_type=jnp.float32, precision=hi)
    e = jnp.dot(ones_b, (bf < bnds + S).astype(jnp.float32),
                preferred_element_type=jnp.float32, precision=hi)
    nch = jnp.where(wvalid, jnp.maximum(e - b0, 1.0), 0.0)

    ii = lax.broadcasted_iota(jnp.int32, (S, S), 0)
    jj = lax.broadcasted_iota(jnp.int32, (S, S), 1)
    u_incl = (ii <= jj).astype(jnp.float32)
    csum = jnp.dot(nch, u_incl, preferred_element_type=jnp.float32,
                   precision=hi)
    offs = csum - nch

    cid = lax.broadcasted_iota(jnp.int32, (CH_PAD, 1), 0).astype(jnp.float32)
    cmp = (csum <= cid).astype(jnp.float32) * wvf
    ones = jnp.ones((S, 1), jnp.float32)
    wofc = jnp.dot(cmp, ones, preferred_element_type=jnp.float32,
                   precision=hi)

    lanef = lax.broadcasted_iota(jnp.int32, (CH_PAD, S), 1).astype(jnp.float32)
    g1 = (lanef == wofc).astype(jnp.float32)
    b0g = jnp.dot(g1 * b0, ones, preferred_element_type=jnp.float32,
                  precision=hi)
    offsg = jnp.dot(g1 * offs, ones, preferred_element_type=jnp.float32,
                    precision=hi)

    local = cid - offsg
    validc = wofc <= float(NW - 1)
    rb = jnp.clip(b0g + local, 0.0, float(NBLK - 1)).astype(jnp.int32)
    flag = jnp.where(validc,
                     jnp.where(local == 0.0, 1, 0),
                     -1).astype(jnp.int32)
    wc = jnp.minimum(wofc, float(NW - 1)).astype(jnp.int32)

    rb_ref[...] = rb
    wc_ref[...] = wc
    fl_ref[...] = flag


def _mlp_seg_kernel(rb_ref, wc_ref, fl_ref,
                    x_ref, idxc_ref, W1_ref, b1_ref, W2_ref, b2_ref,
                    out_ref, h_ref):
    g = pl.program_id(0)
    p = lax.rem(g, 2)

    @pl.when(g > 0)
    def _():
        gc = g - 1
        flag = fl_ref[gc]
        w = wc_ref[gc]
        t = jnp.dot(h_ref[1 - p], W2_ref[...],
                    preferred_element_type=jnp.float32,
                    precision=lax.Precision.HIGHEST)
        t = jnp.maximum(t + b2_ref[...], 0.0)
        local = idxc_ref[0, 0, :] - w * S
        local = jnp.where(flag >= 0, local, -1)
        iota = lax.broadcasted_iota(jnp.int32, (S, R), 0)
        oh = (iota == local[None, :]).astype(jnp.float32)
        part = jnp.dot(oh, t, preferred_element_type=jnp.float32,
                       precision=lax.Precision.HIGHEST)

        @pl.when(flag == 1)
        def _():
            out_ref[...] = part

        @pl.when(flag != 1)
        def _():
            out_ref[...] += part

    @pl.when(g < CHUNKS)
    def _():
        h = jnp.dot(x_ref[...], W1_ref[...], preferred_element_type=jnp.float32,
                    precision=lax.Precision.HIGHEST)
        h_ref[p] = jnp.maximum(h + b1_ref[...], 0.0)


def _head_kernel(seg_ref, W3_ref, b3_ref, W4_ref, b4_ref, out_ref):
    u = jnp.dot(seg_ref[...], W3_ref[...], preferred_element_type=jnp.float32,
                precision=lax.Precision.HIGHEST)
    u = jnp.maximum(u + b3_ref[...], 0.0)
    v = jnp.dot(u, W4_ref[...], preferred_element_type=jnp.float32,
                precision=lax.Precision.HIGHEST)
    out_ref[...] = v + b4_ref[...]


def kernel(x, idx, W1, b1, W2, b2, W3, b3, W4, b4):
    idx32 = idx.astype(jnp.int32)

    idx2d = idx32.reshape(NBLK, R)
    pad = jnp.full((NBLK_PAD - NBLK,), BIG, jnp.int32)
    bf_col = jnp.concatenate([idx2d[:, 0], pad]).reshape(NBLK_PAD, 1)
    bl_col = jnp.concatenate([idx2d[:, R - 1], pad]).reshape(NBLK_PAD, 1)

    rb2, wc2, fl2 = pl.pallas_call(
        _schedule_kernel,
        in_specs=[
            pl.BlockSpec((NBLK_PAD, 1), lambda: (0, 0)),
            pl.BlockSpec((NBLK_PAD, 1), lambda: (0, 0)),
        ],
        out_specs=[
            pl.BlockSpec((CH_PAD, 1), lambda: (0, 0)),
            pl.BlockSpec((CH_PAD, 1), lambda: (0, 0)),
            pl.BlockSpec((CH_PAD, 1), lambda: (0, 0)),
        ],
        out_shape=[jax.ShapeDtypeStruct((CH_PAD, 1), jnp.int32)] * 3,
    )(bf_col, bl_col)
    rb = rb2.reshape(CH_PAD)
    wc = wc2.reshape(CH_PAD)
    fl = fl2.reshape(CH_PAD)

    idx3 = idx32.reshape(NBLK, 1, R)
    b1r = b1.reshape(1, D)
    b2r = b2.reshape(1, D)

    seg = pl.pallas_call(
        _mlp_seg_kernel,
        grid_spec=pltpu.PrefetchScalarGridSpec(
            num_scalar_prefetch=3,
            grid=(CHUNKS + 1,),
            in_specs=[
                pl.BlockSpec(
                    (R, D),
                    lambda g, rb, w, fl: (rb[jnp.minimum(g, CHUNKS - 1)], 0)),
                pl.BlockSpec(
                    (1, 1, R),
                    lambda g, rb, w, fl: (rb[jnp.maximum(g - 1, 0)], 0, 0)),
                pl.BlockSpec((D, D), lambda g, rb, w, fl: (0, 0)),
                pl.BlockSpec((1, D), lambda g, rb, w, fl: (0, 0)),
                pl.BlockSpec((D, D), lambda g, rb, w, fl: (0, 0)),
                pl.BlockSpec((1, D), lambda g, rb, w, fl: (0, 0)),
            ],
            out_specs=pl.BlockSpec(
                (S, D), lambda g, rb, w, fl: (w[jnp.maximum(g - 1, 0)], 0)),
            scratch_shapes=[pltpu.VMEM((2, R, D), jnp.float32)],
        ),
        out_shape=jax.ShapeDtypeStruct((SEGP, D), jnp.float32),
    )(rb, wc, fl, x, idx3, W1, b1r, W2, b2r)

    out = pl.pallas_call(
        _head_kernel,
        in_specs=[
            pl.BlockSpec((SEGP, D), lambda: (0, 0)),
            pl.BlockSpec((D, 20), lambda: (0, 0)),
            pl.BlockSpec((1, 20), lambda: (0, 0)),
            pl.BlockSpec((20, 1), lambda: (0, 0)),
            pl.BlockSpec((1, 1), lambda: (0, 0)),
        ],
        out_specs=pl.BlockSpec((SEGP, 1), lambda: (0, 0)),
        out_shape=jax.ShapeDtypeStruct((SEGP, 1), jnp.float32),
    )(seg, W3, b3.reshape(1, 20), W4, b4.reshape(1, 1))

    return out[:NUM_SEG]


# speedup vs baseline: 1.8602x; 1.8602x over previous
import jax
import jax.numpy as jnp
from jax import lax
from jax.experimental import pallas as pl
from jax.experimental.pallas import tpu as pltpu

N = 320000
D = 128
NUM_SEG = 10000

R = 512
NBLK = N // R
NBLK_PAD = 640
S = 128
NW = (NUM_SEG + S - 1) // S
SEGP = NW * S
CHUNKS = NBLK + 2 * NW
CH_PAD = 896
BIG = 1 << 24


def _schedule_kernel(bf_ref, bl_ref, rb_ref, wc_ref, fl_ref):
    hi = lax.Precision.HIGHEST
    bf = bf_ref[...].astype(jnp.float32)
    bl = bl_ref[...].astype(jnp.float32)
    lane = lax.broadcasted_iota(jnp.int32, (1, S), 1)
    wvalid = lane < NW
    wvf = wvalid.astype(jnp.float32)
    bnds = (lane * S).astype(jnp.float32)

    ones_b = jnp.ones((1, NBLK_PAD), jnp.float32)
    b0 = jnp.dot(ones_b, (bl < bnds).astype(jnp.float32),
                 preferred_element_type=jnp.float32, precision=hi)
    e = jnp.dot(ones_b, (bf < bnds + S).astype(jnp.float32),
                preferred_element_type=jnp.float32, precision=hi)
    nch = jnp.where(wvalid, jnp.maximum(e - b0, 1.0), 0.0)

    ii = lax.broadcasted_iota(jnp.int32, (S, S), 0)
    jj = lax.broadcasted_iota(jnp.int32, (S, S), 1)
    u_incl = (ii <= jj).astype(jnp.float32)
    csum = jnp.dot(nch, u_incl, preferred_element_type=jnp.float32,
                   precision=hi)
    offs = csum - nch

    cid = lax.broadcasted_iota(jnp.int32, (CH_PAD, 1), 0).astype(jnp.float32)
    cmp = (csum <= cid).astype(jnp.float32) * wvf
    ones = jnp.ones((S, 1), jnp.float32)
    wofc = jnp.dot(cmp, ones, preferred_element_type=jnp.float32,
                   precision=hi)

    lanef = lax.broadcasted_iota(jnp.int32, (CH_PAD, S), 1).astype(jnp.float32)
    g1 = (lanef == wofc).astype(jnp.float32)
    b0g = jnp.dot(g1 * b0, ones, preferred_element_type=jnp.float32,
                  precision=hi)
    offsg = jnp.dot(g1 * offs, ones, preferred_element_type=jnp.float32,
                    precision=hi)

    local = cid - offsg
    validc = wofc <= float(NW - 1)
    rb = jnp.clip(b0g + local, 0.0, float(NBLK - 1)).astype(jnp.int32)
    flag = jnp.where(validc,
                     jnp.where(local == 0.0, 1, 0),
                     -1).astype(jnp.int32)
    wc = jnp.minimum(wofc, float(NW - 1)).astype(jnp.int32)

    rb_ref[...] = rb
    wc_ref[...] = wc
    fl_ref[...] = flag


def _mlp_seg_kernel(rb_ref, wc_ref, fl_ref,
                    x_ref, idxc_ref, W1_ref, b1_ref, W2_ref, b2_ref,
                    out_ref, h_ref):
    g = pl.program_id(0)
    p = lax.rem(g, 2)

    @pl.when(g > 0)
    def _():
        gc = g - 1
        flag = fl_ref[gc]
        w = wc_ref[gc]
        t = jnp.dot(h_ref[1 - p], W2_ref[...],
                    preferred_element_type=jnp.float32)
        t = jnp.maximum(t + b2_ref[...], 0.0)
        local = idxc_ref[0, 0, :] - w * S
        local = jnp.where(flag >= 0, local, -1)
        iota = lax.broadcasted_iota(jnp.int32, (S, R), 0)
        oh = (iota == local[None, :]).astype(jnp.bfloat16)
        t_hi = t.astype(jnp.bfloat16)
        t_lo = (t - t_hi.astype(jnp.float32)).astype(jnp.bfloat16)
        part = (jnp.dot(oh, t_hi, preferred_element_type=jnp.float32)
                + jnp.dot(oh, t_lo, preferred_element_type=jnp.float32))

        @pl.when(flag == 1)
        def _():
            out_ref[...] = part

        @pl.when(flag != 1)
        def _():
            out_ref[...] += part

    @pl.when(g < CHUNKS)
    def _():
        h = jnp.dot(x_ref[...], W1_ref[...], preferred_element_type=jnp.float32)
        h_ref[p] = jnp.maximum(h + b1_ref[...], 0.0)


def _head_kernel(seg_ref, W3_ref, b3_ref, W4_ref, b4_ref, out_ref):
    u = jnp.dot(seg_ref[...], W3_ref[...], preferred_element_type=jnp.float32)
    u = jnp.maximum(u + b3_ref[...], 0.0)
    v = jnp.dot(u, W4_ref[...], preferred_element_type=jnp.float32)
    out_ref[...] = v + b4_ref[...]


def kernel(x, idx, W1, b1, W2, b2, W3, b3, W4, b4):
    idx32 = idx.astype(jnp.int32)

    idx2d = idx32.reshape(NBLK, R)
    pad = jnp.full((NBLK_PAD - NBLK,), BIG, jnp.int32)
    bf_col = jnp.concatenate([idx2d[:, 0], pad]).reshape(NBLK_PAD, 1)
    bl_col = jnp.concatenate([idx2d[:, R - 1], pad]).reshape(NBLK_PAD, 1)

    rb2, wc2, fl2 = pl.pallas_call(
        _schedule_kernel,
        in_specs=[
            pl.BlockSpec((NBLK_PAD, 1), lambda: (0, 0)),
            pl.BlockSpec((NBLK_PAD, 1), lambda: (0, 0)),
        ],
        out_specs=[
            pl.BlockSpec((CH_PAD, 1), lambda: (0, 0)),
            pl.BlockSpec((CH_PAD, 1), lambda: (0, 0)),
            pl.BlockSpec((CH_PAD, 1), lambda: (0, 0)),
        ],
        out_shape=[jax.ShapeDtypeStruct((CH_PAD, 1), jnp.int32)] * 3,
    )(bf_col, bl_col)
    rb = rb2.reshape(CH_PAD)
    wc = wc2.reshape(CH_PAD)
    fl = fl2.reshape(CH_PAD)

    idx3 = idx32.reshape(NBLK, 1, R)
    b1r = b1.reshape(1, D)
    b2r = b2.reshape(1, D)

    seg = pl.pallas_call(
        _mlp_seg_kernel,
        grid_spec=pltpu.PrefetchScalarGridSpec(
            num_scalar_prefetch=3,
            grid=(CHUNKS + 1,),
            in_specs=[
                pl.BlockSpec(
                    (R, D),
                    lambda g, rb, w, fl: (rb[jnp.minimum(g, CHUNKS - 1)], 0)),
                pl.BlockSpec(
                    (1, 1, R),
                    lambda g, rb, w, fl: (rb[jnp.maximum(g - 1, 0)], 0, 0)),
                pl.BlockSpec((D, D), lambda g, rb, w, fl: (0, 0)),
                pl.BlockSpec((1, D), lambda g, rb, w, fl: (0, 0)),
                pl.BlockSpec((D, D), lambda g, rb, w, fl: (0, 0)),
                pl.BlockSpec((1, D), lambda g, rb, w, fl: (0, 0)),
            ],
            out_specs=pl.BlockSpec(
                (S, D), lambda g, rb, w, fl: (w[jnp.maximum(g - 1, 0)], 0)),
            scratch_shapes=[pltpu.VMEM((2, R, D), jnp.float32)],
        ),
        out_shape=jax.ShapeDtypeStruct((SEGP, D), jnp.float32),
    )(rb, wc, fl, x, idx3, W1, b1r, W2, b2r)

    out = pl.pallas_call(
        _head_kernel,
        in_specs=[
            pl.BlockSpec((SEGP, D), lambda: (0, 0)),
            pl.BlockSpec((D, 20), lambda: (0, 0)),
            pl.BlockSpec((1, 20), lambda: (0, 0)),
            pl.BlockSpec((20, 1), lambda: (0, 0)),
            pl.BlockSpec((1, 1), lambda: (0, 0)),
        ],
        out_specs=pl.BlockSpec((SEGP, 1), lambda: (0, 0)),
        out_shape=jax.ShapeDtypeStruct((SEGP, 1), jnp.float32),
    )(seg, W3, b3.reshape(1, 20), W4, b4.reshape(1, 1))

    return out[:NUM_SEG]


# speedup vs baseline: 3.7865x; 2.0355x over previous
import jax
import jax.numpy as jnp
from jax import lax
from jax.experimental import pallas as pl
from jax.experimental.pallas import tpu as pltpu

N = 320000
D = 128
NUM_SEG = 10000

R = 2000
NBLK = N // R
NBLK_PAD = 160
S = 256
NW = (NUM_SEG + S - 1) // S
SEGP = NW * S
WL = 128
CHUNKS = NBLK + 2 * NW
CH_PAD = 240
BIG = 1 << 24


def _schedule_kernel(bf_ref, bl_ref, rb_ref, wc_ref, fl_ref):
    hi = lax.Precision.HIGHEST
    bf = bf_ref[...].astype(jnp.float32)
    bl = bl_ref[...].astype(jnp.float32)
    lane = lax.broadcasted_iota(jnp.int32, (1, WL), 1)
    wvalid = lane < NW
    wvf = wvalid.astype(jnp.float32)
    bnds = (lane * S).astype(jnp.float32)

    ones_b = jnp.ones((1, NBLK_PAD), jnp.float32)
    b0 = jnp.dot(ones_b, (bl < bnds).astype(jnp.float32),
                 preferred_element_type=jnp.float32, precision=hi)
    e = jnp.dot(ones_b, (bf < bnds + S).astype(jnp.float32),
                preferred_element_type=jnp.float32, precision=hi)
    nch = jnp.where(wvalid, jnp.maximum(e - b0, 1.0), 0.0)

    ii = lax.broadcasted_iota(jnp.int32, (WL, WL), 0)
    jj = lax.broadcasted_iota(jnp.int32, (WL, WL), 1)
    u_incl = (ii <= jj).astype(jnp.float32)
    csum = jnp.dot(nch, u_incl, preferred_element_type=jnp.float32,
                   precision=hi)
    offs = csum - nch

    cid = lax.broadcasted_iota(jnp.int32, (CH_PAD, 1), 0).astype(jnp.float32)
    cmp = (csum <= cid).astype(jnp.float32) * wvf
    ones = jnp.ones((WL, 1), jnp.float32)
    wofc = jnp.dot(cmp, ones, preferred_element_type=jnp.float32,
                   precision=hi)

    lanef = lax.broadcasted_iota(jnp.int32, (CH_PAD, WL), 1).astype(jnp.float32)
    g1 = (lanef == wofc).astype(jnp.float32)
    b0g = jnp.dot(g1 * b0, ones, preferred_element_type=jnp.float32,
                  precision=hi)
    offsg = jnp.dot(g1 * offs, ones, preferred_element_type=jnp.float32,
                    precision=hi)

    local = cid - offsg
    validc = wofc <= float(NW - 1)
    rb = jnp.clip(b0g + local, 0.0, float(NBLK - 1)).astype(jnp.int32)
    flag = jnp.where(validc,
                     jnp.where(local == 0.0, 1, 0),
                     -1).astype(jnp.int32)
    wc = jnp.minimum(wofc, float(NW - 1)).astype(jnp.int32)

    rb_ref[...] = rb
    wc_ref[...] = wc
    fl_ref[...] = flag


def _mlp_seg_kernel(rb_ref, wc_ref, fl_ref,
                    x_ref, idxc_ref, W1_ref, b1_ref, W2_ref, b2_ref,
                    out_ref, h_ref):
    g = pl.program_id(0)
    p = lax.rem(g, 2)

    @pl.when(g > 0)
    def _():
        gc = g - 1
        flag = fl_ref[gc]
        w = wc_ref[gc]
        t = jnp.dot(h_ref[1 - p], W2_ref[...],
                    preferred_element_type=jnp.float32)
        t = jnp.maximum(t + b2_ref[...], 0.0)
        local = idxc_ref[0, 0, :] - w * S
        local = jnp.where(flag >= 0, local, -1)
        iota = lax.broadcasted_iota(jnp.int32, (S, R), 0)
        oh = (iota == local[None, :]).astype(jnp.bfloat16)
        t_hi = t.astype(jnp.bfloat16)
        t_lo = (t - t_hi.astype(jnp.float32)).astype(jnp.bfloat16)
        tcat = jnp.concatenate([t_hi, t_lo], axis=1)
        p2 = jnp.dot(oh, tcat, preferred_element_type=jnp.float32)
        part = p2[:, :D] + p2[:, D:]

        @pl.when(flag == 1)
        def _():
            out_ref[...] = part

        @pl.when(flag != 1)
        def _():
            out_ref[...] += part

    @pl.when(g < CHUNKS)
    def _():
        h = jnp.dot(x_ref[...], W1_ref[...], preferred_element_type=jnp.float32)
        h_ref[p] = jnp.maximum(h + b1_ref[...], 0.0)


def _head_kernel(seg_ref, W3_ref, b3_ref, W4_ref, b4_ref, out_ref):
    u = jnp.dot(seg_ref[...], W3_ref[...], preferred_element_type=jnp.float32)
    u = jnp.maximum(u + b3_ref[...], 0.0)
    v = jnp.dot(u, W4_ref[...], preferred_element_type=jnp.float32)
    out_ref[...] = v + b4_ref[...]


def kernel(x, idx, W1, b1, W2, b2, W3, b3, W4, b4):
    idx32 = idx.astype(jnp.int32)

    idx2d = idx32.reshape(NBLK, R)
    pad = jnp.full((NBLK_PAD - NBLK,), BIG, jnp.int32)
    bf_col = jnp.concatenate([idx2d[:, 0], pad]).reshape(NBLK_PAD, 1)
    bl_col = jnp.concatenate([idx2d[:, R - 1], pad]).reshape(NBLK_PAD, 1)

    rb2, wc2, fl2 = pl.pallas_call(
        _schedule_kernel,
        in_specs=[
            pl.BlockSpec((NBLK_PAD, 1), lambda: (0, 0)),
            pl.BlockSpec((NBLK_PAD, 1), lambda: (0, 0)),
        ],
        out_specs=[
            pl.BlockSpec((CH_PAD, 1), lambda: (0, 0)),
            pl.BlockSpec((CH_PAD, 1), lambda: (0, 0)),
            pl.BlockSpec((CH_PAD, 1), lambda: (0, 0)),
        ],
        out_shape=[jax.ShapeDtypeStruct((CH_PAD, 1), jnp.int32)] * 3,
    )(bf_col, bl_col)
    rb = rb2.reshape(CH_PAD)
    wc = wc2.reshape(CH_PAD)
    fl = fl2.reshape(CH_PAD)

    idx3 = idx32.reshape(NBLK, 1, R)
    b1r = b1.reshape(1, D)
    b2r = b2.reshape(1, D)

    seg = pl.pallas_call(
        _mlp_seg_kernel,
        grid_spec=pltpu.PrefetchScalarGridSpec(
            num_scalar_prefetch=3,
            grid=(CHUNKS + 1,),
            in_specs=[
                pl.BlockSpec(
                    (R, D),
                    lambda g, rb, w, fl: (rb[jnp.minimum(g, CHUNKS - 1)], 0)),
                pl.BlockSpec(
                    (1, 1, R),
                    lambda g, rb, w, fl: (rb[jnp.maximum(g - 1, 0)], 0, 0)),
                pl.BlockSpec((D, D), lambda g, rb, w, fl: (0, 0)),
                pl.BlockSpec((1, D), lambda g, rb, w, fl: (0, 0)),
                pl.BlockSpec((D, D), lambda g, rb, w, fl: (0, 0)),
                pl.BlockSpec((1, D), lambda g, rb, w, fl: (0, 0)),
            ],
            out_specs=pl.BlockSpec(
                (S, D), lambda g, rb, w, fl: (w[jnp.maximum(g - 1, 0)], 0)),
            scratch_shapes=[pltpu.VMEM((2, R, D), jnp.float32)],
        ),
        out_shape=jax.ShapeDtypeStruct((SEGP, D), jnp.float32),
    )(rb, wc, fl, x, idx3, W1, b1r, W2, b2r)

    out = pl.pallas_call(
        _head_kernel,
        in_specs=[
            pl.BlockSpec((SEGP, D), lambda: (0, 0)),
            pl.BlockSpec((D, 20), lambda: (0, 0)),
            pl.BlockSpec((1, 20), lambda: (0, 0)),
            pl.BlockSpec((20, 1), lambda: (0, 0)),
            pl.BlockSpec((1, 1), lambda: (0, 0)),
        ],
        out_specs=pl.BlockSpec((SEGP, 1), lambda: (0, 0)),
        out_shape=jax.ShapeDtypeStruct((SEGP, 1), jnp.float32),
    )(seg, W3, b3.reshape(1, 20), W4, b4.reshape(1, 1))

    return out[:NUM_SEG]
